# baseline (speedup 1.0000x reference)
import jax
import jax.numpy as jnp
from jax import lax
from jax.experimental import pallas as pl
from jax.experimental.pallas import tpu as pltpu
from jax.experimental.pallas import tpu_sc as plsc

N = 10000
D = 128
NP = 10240
DUMMY = N
NC, NS, CB = 2, 16, 128
CBP = 64
RPT = NP // NS
BK = 1024

_mesh = plsc.VectorSubcoreMesh(
    core_axis_name="c", subcore_axis_name="s", num_cores=NC, num_subcores=NS)
_no_layout = pltpu.CompilerParams(needs_layout_passes=False)


def _fill_zero(ref, rows):
  v = jnp.zeros((16,), jnp.float32)
  w = ref.shape[1]

  def row(i, _):
    for g in range(w // 16):
      ref[i, pl.ds(g * 16, 16)] = v
    return 0

  lax.fori_loop(0, rows, row, 0)


def _fill_zero1(ref):
  v = jnp.zeros((16,), jnp.float32)

  def row(i, _):
    ref[pl.ds(i * 16, 16)] = v
    return 0

  lax.fori_loop(0, ref.shape[0] // 16, row, 0)


def _zero_acc(acc_s, buf, s):
  rows = buf.shape[0]
  _fill_zero(buf, rows)
  for k in range(RPT // rows):
    pltpu.sync_copy(buf, acc_s.at[pl.ds(s * RPT + k * rows, rows)])


def _scal1_body(gr, rows3, cols3, out_o,
                ridx, cidx, grt, adeg, aagg, aod):
  c = lax.axis_index("c")
  s = lax.axis_index("s")
  pltpu.sync_copy(gr, grt)
  _fill_zero1(adeg)
  _fill_zero1(aagg)
  _fill_zero1(aod)
  nch = rows3.shape[2]
  ones16 = jnp.ones((16,), jnp.float32)

  def chunk(j, _):
    pltpu.sync_copy(rows3.at[c, s, j], ridx)
    pltpu.sync_copy(cols3.at[c, s, j], cidx)

    def grp(k, _):
      r16 = ridx[pl.ds(k * 16, 16)]
      c16 = cidx[pl.ds(k * 16, 16)]
      g = plsc.load_gather(grt, [r16])
      plsc.addupdate_scatter(aagg, [c16], g)
      plsc.addupdate_scatter(adeg, [c16], ones16)
      plsc.addupdate_scatter(aod, [r16], ones16)
      return 0

    lax.fori_loop(0, CB // 16, grp, 0)
    return 0

  lax.fori_loop(0, nch, chunk, 0)
  base = ((c * NS) + s) * 3
  pltpu.sync_copy(adeg, out_o.at[pl.ds((base + 0) * NP, NP)])
  pltpu.sync_copy(aagg, out_o.at[pl.ds((base + 1) * NP, NP)])
  pltpu.sync_copy(aod, out_o.at[pl.ds((base + 2) * NP, NP)])


def _scal1(gr, rows3, cols3):
  f = pl.kernel(
      _scal1_body,
      out_type=jax.ShapeDtypeStruct((NC * NS * 3 * NP,), jnp.float32),
      mesh=_mesh,
      compiler_params=_no_layout,
      scratch_types=[
          pltpu.VMEM((CB,), jnp.int32), pltpu.VMEM((CB,), jnp.int32),
          pltpu.VMEM((NP,), jnp.float32),
          pltpu.VMEM((NP,), jnp.float32), pltpu.VMEM((NP,), jnp.float32),
          pltpu.VMEM((NP,), jnp.float32),
      ])
  return f(gr, rows3, cols3)


def _scal2_body(uk, uq, rows3, cols3, out_o,
                ridx, cidx, ukt, uqt, adk, adq):
  c = lax.axis_index("c")
  s = lax.axis_index("s")
  pltpu.sync_copy(uk, ukt)
  pltpu.sync_copy(uq, uqt)
  _fill_zero1(adk)
  _fill_zero1(adq)
  nch = rows3.shape[2]

  def chunk(j, _):
    pltpu.sync_copy(rows3.at[c, s, j], ridx)
    pltpu.sync_copy(cols3.at[c, s, j], cidx)

    def grp(k, _):
      r16 = ridx[pl.ds(k * 16, 16)]
      c16 = cidx[pl.ds(k * 16, 16)]
      plsc.addupdate_scatter(adk, [r16], plsc.load_gather(ukt, [c16]))
      plsc.addupdate_scatter(adq, [r16], plsc.load_gather(uqt, [c16]))
      return 0

    lax.fori_loop(0, CB // 16, grp, 0)
    return 0

  lax.fori_loop(0, nch, chunk, 0)
  base = ((c * NS) + s) * 2
  pltpu.sync_copy(adk, out_o.at[pl.ds((base + 0) * NP, NP)])
  pltpu.sync_copy(adq, out_o.at[pl.ds((base + 1) * NP, NP)])


def _scal2(uk, uq, rows3, cols3):
  f = pl.kernel(
      _scal2_body,
      out_type=jax.ShapeDtypeStruct((NC * NS * 2 * NP,), jnp.float32),
      mesh=_mesh,
      compiler_params=_no_layout,
      scratch_types=[
          pltpu.VMEM((CB,), jnp.int32), pltpu.VMEM((CB,), jnp.int32),
          pltpu.VMEM((NP,), jnp.float32), pltpu.VMEM((NP,), jnp.float32),
          pltpu.VMEM((NP,), jnp.float32), pltpu.VMEM((NP,), jnp.float32),
      ])
  return f(uk, uq, rows3, cols3)


def _gs_body(table, idxa3, idxb3, acc_o, aidx, bidx, rows_v, acc_s, sem):
  c = lax.axis_index("c")
  s = lax.axis_index("s")
  _zero_acc(acc_s, rows_v, s)
  plsc.subcore_barrier()
  nch = idxa3.shape[2]

  def chunk(j, _):
    pltpu.sync_copy(idxa3.at[c, s, j], aidx)
    pltpu.sync_copy(idxb3.at[c, s, j], bidx)
    pltpu.async_copy(table.at[aidx], rows_v, sem).wait()
    pltpu.sync_copy(rows_v, acc_s.at[bidx], add=True)
    return 0

  lax.fori_loop(0, nch, chunk, 0)
  plsc.subcore_barrier()
  pltpu.sync_copy(acc_s.at[pl.ds(s * RPT, RPT)], acc_o.at[c, pl.ds(s * RPT, RPT)])


def _gs(table, idxa3, idxb3):
  f = pl.kernel(
      _gs_body,
      out_type=jax.ShapeDtypeStruct((NC, NP, D), jnp.float32),
      mesh=_mesh,
      scratch_types=[
          pltpu.VMEM((CB,), jnp.int32), pltpu.VMEM((CB,), jnp.int32),
          pltpu.VMEM((CB, D), jnp.float32),
          pltpu.VMEM_SHARED((NP, D), jnp.float32),
          pltpu.SemaphoreType.DMA,
      ])
  return f(table, idxa3, idxb3)


def _p2_run(table, acc_s, rows2, cols2, ridx, cidx, rows_v, sem, s):
  nch = rows2.shape[1]

  def chunk(j, _):
    pltpu.sync_copy(cols2.at[s, j], cidx)
    pltpu.sync_copy(rows2.at[s, j], ridx)
    pltpu.async_copy(table.at[cidx], rows_v, sem).wait()
    pltpu.sync_copy(rows_v, acc_s.at[ridx], add=True)
    return 0

  lax.fori_loop(0, nch, chunk, 0)


def _p2_body(tk, tq, rows2, cols2, acc_o, ridx, cidx, rows_v, acc_s, sem):
  c = lax.axis_index("c")
  s = lax.axis_index("s")
  _zero_acc(acc_s, rows_v, s)
  plsc.subcore_barrier()

  @pl.when(c == 0)
  def _():
    _p2_run(tk, acc_s, rows2, cols2, ridx, cidx, rows_v, sem, s)

  @pl.when(c == 1)
  def _():
    _p2_run(tq, acc_s, rows2, cols2, ridx, cidx, rows_v, sem, s)

  plsc.subcore_barrier()
  pltpu.sync_copy(acc_s.at[pl.ds(s * RPT, RPT)], acc_o.at[c, pl.ds(s * RPT, RPT)])


def _p2(tk, tq, rows2, cols2):
  f = pl.kernel(
      _p2_body,
      out_type=jax.ShapeDtypeStruct((NC, NP, D), jnp.float32),
      mesh=_mesh,
      scratch_types=[
          pltpu.VMEM((CB,), jnp.int32), pltpu.VMEM((CB,), jnp.int32),
          pltpu.VMEM((CB, D), jnp.float32),
          pltpu.VMEM_SHARED((NP, D), jnp.float32),
          pltpu.SemaphoreType.DMA,
      ])
  return f(tk, tq, rows2, cols2)


def _p4_body(xrq, xt, rows4, cols4, ss_o,
             ridx, cidx, a_v, b_v, cbuf, acc_s, sem, sem2):
  c = lax.axis_index("c")
  s = lax.axis_index("s")
  _zero_acc(acc_s, cbuf, s)
  plsc.subcore_barrier()
  nch = rows4.shape[2]

  def chunk(j, _):
    pltpu.sync_copy(rows4.at[c, s, j], ridx)
    pltpu.sync_copy(cols4.at[c, s, j], cidx)
    d1 = pltpu.async_copy(xrq.at[ridx], a_v, sem)
    d2 = pltpu.async_copy(xt.at[cidx], b_v, sem2)
    d1.wait()
    d2.wait()

    def edge(i, _):
      for g in range(D // 16):
        sl = pl.ds(g * 16, 16)
        cbuf[i, sl] = jnp.abs(a_v[i, sl] - b_v[i, sl])
      return 0

    lax.fori_loop(0, CBP, edge, 0)
    pltpu.sync_copy(cbuf, acc_s.at[ridx], add=True)
    return 0

  lax.fori_loop(0, nch, chunk, 0)
  plsc.subcore_barrier()
  pltpu.sync_copy(acc_s.at[pl.ds(s * RPT, RPT)], ss_o.at[c, pl.ds(s * RPT, RPT)])


def _p4(xrq, xt, rows4, cols4):
  f = pl.kernel(
      _p4_body,
      out_type=jax.ShapeDtypeStruct((NC, NP, D), jnp.float32),
      mesh=_mesh,
      scratch_types=[
          pltpu.VMEM((CBP,), jnp.int32), pltpu.VMEM((CBP,), jnp.int32),
          pltpu.VMEM((CBP, D), jnp.float32), pltpu.VMEM((CBP, D), jnp.float32),
          pltpu.VMEM((CBP, D), jnp.float32),
          pltpu.VMEM_SHARED((NP, D), jnp.float32),
          pltpu.SemaphoreType.DMA, pltpu.SemaphoreType.DMA,
      ])
  return f(xrq, xt, rows4, cols4)


def _k1_body(x_ref, wg_ref, wr_ref, wroot_ref, bgr_ref, xw_ref, gr_ref, fit_ref):
  xb = x_ref[...]
  xw = jnp.dot(xb, wg_ref[...], preferred_element_type=jnp.float32)
  gr = jnp.dot(xb, wr_ref[...], preferred_element_type=jnp.float32)
  groot = jnp.dot(xb, wroot_ref[...], preferred_element_type=jnp.float32)
  xw_ref[...] = xw
  gr_ref[...] = gr
  fit_ref[...] = gr + groot + bgr_ref[0, 0]


def _k1(xp, w_gcn, wg_rel, wg_root, bg_rel):
  grid = NP // BK
  return pl.pallas_call(
      _k1_body,
      grid=(grid,),
      in_specs=[
          pl.BlockSpec((BK, D), lambda i: (i, 0)),
          pl.BlockSpec((D, D), lambda i: (0, 0)),
          pl.BlockSpec((D, 1), lambda i: (0, 0)),
          pl.BlockSpec((D, 1), lambda i: (0, 0)),
          pl.BlockSpec((1, 1), lambda i: (0, 0)),
      ],
      out_specs=[
          pl.BlockSpec((BK, D), lambda i: (i, 0)),
          pl.BlockSpec((BK, 1), lambda i: (i, 0)),
          pl.BlockSpec((BK, 1), lambda i: (i, 0)),
      ],
      out_shape=[
          jax.ShapeDtypeStruct((NP, D), jnp.float32),
          jax.ShapeDtypeStruct((NP, 1), jnp.float32),
          jax.ShapeDtypeStruct((NP, 1), jnp.float32),
      ])(xp, w_gcn, wg_rel, wg_root, bg_rel)


def _k2_body(a1_ref, xw_ref, fit_ref, y_ref, dinv_ref, fitness_ref):
  a = a1_ref[...]
  deg = 1.0 + jnp.sum(a[:, :, 0, :], axis=(0, 1))[:, None]
  agg = jnp.sum(a[:, :, 1, :], axis=(0, 1))[:, None]
  dinv = lax.rsqrt(deg)
  y_ref[...] = dinv * xw_ref[...]
  dinv_ref[...] = dinv
  fitness_ref[...] = jax.nn.sigmoid(agg + fit_ref[...])


def _k2(s1, xw, fit):
  grid = NP // BK
  return pl.pallas_call(
      _k2_body,
      grid=(grid,),
      in_specs=[
          pl.BlockSpec((NC, NS, 3, BK), lambda i: (0, 0, 0, i)),
          pl.BlockSpec((BK, D), lambda i: (i, 0)),
          pl.BlockSpec((BK, 1), lambda i: (i, 0)),
      ],
      out_specs=[
          pl.BlockSpec((BK, D), lambda i: (i, 0)),
          pl.BlockSpec((BK, 1), lambda i: (i, 0)),
          pl.BlockSpec((BK, 1), lambda i: (i, 0)),
      ],
      out_shape=[
          jax.ShapeDtypeStruct((NP, D), jnp.float32),
          jax.ShapeDtypeStruct((NP, 1), jnp.float32),
          jax.ShapeDtypeStruct((NP, 1), jnp.float32),
      ])(s1, xw, fit)


def _leaky(z):
  return jnp.where(z >= 0, z, 0.01 * z)


def _k3_body(t_ref, dinv_ref, xw_ref, bgcn_ref, wk_ref, bk_ref, wq_ref, bq_ref,
             xt_ref, tk_ref, tq_ref, uk_ref, uq_ref):
  t = t_ref[...]
  dinv = dinv_ref[...]
  xt = dinv * (t[0] + t[1]) + (dinv * dinv) * xw_ref[...] + bgcn_ref[...]
  xt_ref[...] = xt
  uk = jnp.exp(_leaky(jnp.dot(xt, wk_ref[...], preferred_element_type=jnp.float32)
                      + bk_ref[0, 0]))
  uq = jnp.exp(_leaky(jnp.dot(xt, wq_ref[...], preferred_element_type=jnp.float32)
                      + bq_ref[0, 0]))
  tk_ref[...] = uk * xt
  tq_ref[...] = uq * xt
  uk_ref[...] = uk
  uq_ref[...] = uq


def _k3(t, dinv, xw, bgcn, wk, bk, wq, bq):
  grid = NP // BK
  return pl.pallas_call(
      _k3_body,
      grid=(grid,),
      in_specs=[
          pl.BlockSpec((NC, BK, D), lambda i: (0, i, 0)),
          pl.BlockSpec((BK, 1), lambda i: (i, 0)),
          pl.BlockSpec((BK, D), lambda i: (i, 0)),
          pl.BlockSpec((1, D), lambda i: (0, 0)),
          pl.BlockSpec((D, 1), lambda i: (0, 0)),
          pl.BlockSpec((1, 1), lambda i: (0, 0)),
          pl.BlockSpec((D, 1), lambda i: (0, 0)),
          pl.BlockSpec((1, 1), lambda i: (0, 0)),
      ],
      out_specs=[
          pl.BlockSpec((BK, D), lambda i: (i, 0)),
          pl.BlockSpec((BK, D), lambda i: (i, 0)),
          pl.BlockSpec((BK, D), lambda i: (i, 0)),
          pl.BlockSpec((BK, 1), lambda i: (i, 0)),
          pl.BlockSpec((BK, 1), lambda i: (i, 0)),
      ],
      out_shape=[
          jax.ShapeDtypeStruct((NP, D), jnp.float32),
          jax.ShapeDtypeStruct((NP, D), jnp.float32),
          jax.ShapeDtypeStruct((NP, D), jnp.float32),
          jax.ShapeDtypeStruct((NP, 1), jnp.float32),
          jax.ShapeDtypeStruct((NP, 1), jnp.float32),
      ])(t, dinv, xw, bgcn, wk, bk, wq, bq)


def _k4_body(p_ref, s2_ref, tk_ref, tq_ref, uk_ref, uq_ref, xt_ref,
             xrq_ref, kerr_ref):
  p = p_ref[...]
  s2 = s2_ref[...]
  tk = tk_ref[...]
  tq = tq_ref[...]
  xt = xt_ref[...]
  dk = jnp.sum(s2[:, :, 0, :], axis=(0, 1))[:, None] + uk_ref[...]
  dq = jnp.sum(s2[:, :, 1, :], axis=(0, 1))[:, None] + uq_ref[...]
  xr_k = (p[0, :, :] + tk) / dk
  xr_q = (p[1, :, :] + tq) / dq
  xrq_ref[...] = xr_q
  kerr_ref[...] = jnp.sum(jnp.abs(xr_k - xt), axis=1, keepdims=True)


def _k4(p2acc, s2, tk, tq, uk, uq, xt):
  grid = NP // BK
  return pl.pallas_call(
      _k4_body,
      grid=(grid,),
      in_specs=[
          pl.BlockSpec((NC, BK, D), lambda i: (0, i, 0)),
          pl.BlockSpec((NC, NS, 2, BK), lambda i: (0, 0, 0, i)),
          pl.BlockSpec((BK, D), lambda i: (i, 0)),
          pl.BlockSpec((BK, D), lambda i: (i, 0)),
          pl.BlockSpec((BK, 1), lambda i: (i, 0)),
          pl.BlockSpec((BK, 1), lambda i: (i, 0)),
          pl.BlockSpec((BK, D), lambda i: (i, 0)),
      ],
      out_specs=[
          pl.BlockSpec((BK, D), lambda i: (i, 0)),
          pl.BlockSpec((BK, 1), lambda i: (i, 0)),
      ],
      out_shape=[
          jax.ShapeDtypeStruct((NP, D), jnp.float32),
          jax.ShapeDtypeStruct((NP, 1), jnp.float32),
      ])(p2acc, s2, tk, tq, uk, uq, xt)


def _k5_body(s1_ref, kerr_ref, ss_ref, fitness_ref, out_ref):
  s1 = s1_ref[...]
  ss = ss_ref[...]
  outdeg = jnp.sum(s1[:, :, 2, :], axis=(0, 1))[:, None]
  sstot = jnp.sum(ss[0] + ss[1], axis=1, keepdims=True)
  out_ref[...] = fitness_ref[...] - 0.1 * (outdeg * kerr_ref[...] - sstot)


def _k5(s1, kerr, ssacc, fitness):
  grid = NP // BK
  return pl.pallas_call(
      _k5_body,
      grid=(grid,),
      in_specs=[
          pl.BlockSpec((NC, NS, 3, BK), lambda i: (0, 0, 0, i)),
          pl.BlockSpec((BK, 1), lambda i: (i, 0)),
          pl.BlockSpec((NC, BK, D), lambda i: (0, i, 0)),
          pl.BlockSpec((BK, 1), lambda i: (i, 0)),
      ],
      out_specs=pl.BlockSpec((BK, 1), lambda i: (i, 0)),
      out_shape=jax.ShapeDtypeStruct((NP, 1), jnp.float32),
      )(s1, kerr, ssacc, fitness)


def _pad_edges_split(r, c, e, cb=CB):
  tot = NC * NS * cb
  ch = -(-e // tot)
  ch += ch % 2
  ea = tot * ch
  rp = jnp.full((ea,), DUMMY, jnp.int32).at[:e].set(r)
  cp = jnp.full((ea,), DUMMY, jnp.int32).at[:e].set(c)
  return (rp.reshape(NC, NS, ch, cb), cp.reshape(NC, NS, ch, cb))


def _pad_edges_full(r, c, e):
  tot = NS * CB
  ch = -(-e // tot)
  ea = tot * ch
  rp = jnp.full((ea,), DUMMY, jnp.int32).at[:e].set(r)
  cp = jnp.full((ea,), DUMMY, jnp.int32).at[:e].set(c)
  return (rp.reshape(NS, ch, CB), cp.reshape(NS, ch, CB))


@jax.jit
def kernel(x, edge_index, W_gcn, b_gcn, W_key, b_key, W_query, b_query,
           Wg_rel, bg_rel, Wg_root):
  e = edge_index.shape[1]
  row = edge_index[0].astype(jnp.int32)
  col = edge_index[1].astype(jnp.int32)
  rows3, cols3 = _pad_edges_split(row, col, e)
  rows4, cols4 = _pad_edges_split(row, col, e, CBP)
  rows2, cols2 = _pad_edges_full(row, col, e)

  xp = jnp.pad(x, ((0, NP - N), (0, 0)))
  bgr = bg_rel.reshape(1, 1)
  bkk = b_key.reshape(1, 1)
  bqq = b_query.reshape(1, 1)
  bgcn = b_gcn.reshape(1, D)

  xw, gr1, fit = _k1(xp, W_gcn, Wg_rel, Wg_root, bgr)
  s1 = _scal1(gr1.reshape(-1), rows3, cols3).reshape(NC, NS, 3, NP)
  y, dinv, fitness = _k2(s1, xw, fit)
  t = _gs(y, rows3, cols3)
  xt, tk, tq, uk, uq = _k3(t, dinv, xw, bgcn, W_key, bkk, W_query, bqq)
  p2acc = _p2(tk, tq, rows2, cols2)
  s2 = _scal2(uk.reshape(-1), uq.reshape(-1),
              rows3, cols3).reshape(NC, NS, 2, NP)
  xrq, kerr = _k4(p2acc, s2, tk, tq, uk, uq, xt)
  ssacc = _p4(xrq, xt, rows4, cols4)
  return _k5(s1, kerr, ssacc, fitness).reshape(-1)[:N]


# speedup vs baseline: 1.3261x; 1.3261x over previous
import jax
import jax.numpy as jnp
from jax import lax
from jax.experimental import pallas as pl
from jax.experimental.pallas import tpu as pltpu
from jax.experimental.pallas import tpu_sc as plsc

N = 10000
D = 128
NP = 10240
DUMMY = N
NC, NS, CB = 2, 16, 128
CBP = 64
RPT = NP // NS
BK = 1024

_mesh = plsc.VectorSubcoreMesh(
    core_axis_name="c", subcore_axis_name="s", num_cores=NC, num_subcores=NS)
_no_layout = pltpu.CompilerParams(needs_layout_passes=False)


def _fill_zero(ref, rows):
  v = jnp.zeros((16,), jnp.float32)
  w = ref.shape[1]

  def row(i, _):
    for g in range(w // 16):
      ref[i, pl.ds(g * 16, 16)] = v
    return 0

  lax.fori_loop(0, rows, row, 0)


def _fill_zero1(ref):
  v = jnp.zeros((16,), jnp.float32)

  def row(i, _):
    ref[pl.ds(i * 16, 16)] = v
    return 0

  lax.fori_loop(0, ref.shape[0] // 16, row, 0)


def _zero_acc(acc_s, buf, s):
  rows = buf.shape[0]
  _fill_zero(buf, rows)
  for k in range(RPT // rows):
    pltpu.sync_copy(buf, acc_s.at[pl.ds(s * RPT + k * rows, rows)])


def _scal1_body(gr, rows3, cols3, out_o,
                ridx, cidx, grt, adeg, aagg, aod):
  c = lax.axis_index("c")
  s = lax.axis_index("s")
  pltpu.sync_copy(gr, grt)
  _fill_zero1(adeg)
  _fill_zero1(aagg)
  _fill_zero1(aod)
  nch = rows3.shape[2]
  ones16 = jnp.ones((16,), jnp.float32)

  def chunk(j, _):
    pltpu.sync_copy(rows3.at[c, s, j], ridx)
    pltpu.sync_copy(cols3.at[c, s, j], cidx)

    def grp(k, _):
      r16 = ridx[pl.ds(k * 16, 16)]
      c16 = cidx[pl.ds(k * 16, 16)]
      g = plsc.load_gather(grt, [r16])
      plsc.addupdate_scatter(aagg, [c16], g)
      plsc.addupdate_scatter(adeg, [c16], ones16)
      plsc.addupdate_scatter(aod, [r16], ones16)
      return 0

    lax.fori_loop(0, CB // 16, grp, 0)
    return 0

  lax.fori_loop(0, nch, chunk, 0)
  base = ((c * NS) + s) * 3
  pltpu.sync_copy(adeg, out_o.at[pl.ds((base + 0) * NP, NP)])
  pltpu.sync_copy(aagg, out_o.at[pl.ds((base + 1) * NP, NP)])
  pltpu.sync_copy(aod, out_o.at[pl.ds((base + 2) * NP, NP)])


def _scal1(gr, rows3, cols3):
  f = pl.kernel(
      _scal1_body,
      out_type=jax.ShapeDtypeStruct((NC * NS * 3 * NP,), jnp.float32),
      mesh=_mesh,
      compiler_params=_no_layout,
      scratch_types=[
          pltpu.VMEM((CB,), jnp.int32), pltpu.VMEM((CB,), jnp.int32),
          pltpu.VMEM((NP,), jnp.float32),
          pltpu.VMEM((NP,), jnp.float32), pltpu.VMEM((NP,), jnp.float32),
          pltpu.VMEM((NP,), jnp.float32),
      ])
  return f(gr, rows3, cols3)


def _scal2_body(uk, uq, rows3, cols3, out_o,
                ridx, cidx, ukt, uqt, adk, adq):
  c = lax.axis_index("c")
  s = lax.axis_index("s")
  pltpu.sync_copy(uk, ukt)
  pltpu.sync_copy(uq, uqt)
  _fill_zero1(adk)
  _fill_zero1(adq)
  nch = rows3.shape[2]

  def chunk(j, _):
    pltpu.sync_copy(rows3.at[c, s, j], ridx)
    pltpu.sync_copy(cols3.at[c, s, j], cidx)

    def grp(k, _):
      r16 = ridx[pl.ds(k * 16, 16)]
      c16 = cidx[pl.ds(k * 16, 16)]
      plsc.addupdate_scatter(adk, [r16], plsc.load_gather(ukt, [c16]))
      plsc.addupdate_scatter(adq, [r16], plsc.load_gather(uqt, [c16]))
      return 0

    lax.fori_loop(0, CB // 16, grp, 0)
    return 0

  lax.fori_loop(0, nch, chunk, 0)
  base = ((c * NS) + s) * 2
  pltpu.sync_copy(adk, out_o.at[pl.ds((base + 0) * NP, NP)])
  pltpu.sync_copy(adq, out_o.at[pl.ds((base + 1) * NP, NP)])


def _scal2(uk, uq, rows3, cols3):
  f = pl.kernel(
      _scal2_body,
      out_type=jax.ShapeDtypeStruct((NC * NS * 2 * NP,), jnp.float32),
      mesh=_mesh,
      compiler_params=_no_layout,
      scratch_types=[
          pltpu.VMEM((CB,), jnp.int32), pltpu.VMEM((CB,), jnp.int32),
          pltpu.VMEM((NP,), jnp.float32), pltpu.VMEM((NP,), jnp.float32),
          pltpu.VMEM((NP,), jnp.float32), pltpu.VMEM((NP,), jnp.float32),
      ])
  return f(uk, uq, rows3, cols3)


def _gs_body(table, idxa3, idxb3, acc_o, aidx, bidx, rows_v, acc_s, sem):
  c = lax.axis_index("c")
  s = lax.axis_index("s")
  _zero_acc(acc_s, rows_v, s)
  plsc.subcore_barrier()
  nch = idxa3.shape[2]

  def chunk(j, _):
    pltpu.sync_copy(idxa3.at[c, s, j], aidx)
    pltpu.sync_copy(idxb3.at[c, s, j], bidx)
    pltpu.async_copy(table.at[aidx], rows_v, sem).wait()
    pltpu.sync_copy(rows_v, acc_s.at[bidx], add=True)
    return 0

  lax.fori_loop(0, nch, chunk, 0)
  plsc.subcore_barrier()
  pltpu.sync_copy(acc_s.at[pl.ds(s * RPT, RPT)], acc_o.at[c, pl.ds(s * RPT, RPT)])


def _gs(table, idxa3, idxb3):
  f = pl.kernel(
      _gs_body,
      out_type=jax.ShapeDtypeStruct((NC, NP, D), jnp.float32),
      mesh=_mesh,
      scratch_types=[
          pltpu.VMEM((CB,), jnp.int32), pltpu.VMEM((CB,), jnp.int32),
          pltpu.VMEM((CB, D), jnp.float32),
          pltpu.VMEM_SHARED((NP, D), jnp.float32),
          pltpu.SemaphoreType.DMA,
      ])
  return f(table, idxa3, idxb3)


def _p2_run(table, acc_s, rows2, cols2, ridx, cidx, rows_v, sem, s):
  nch = rows2.shape[1]

  def chunk(j, _):
    pltpu.sync_copy(cols2.at[s, j], cidx)
    pltpu.sync_copy(rows2.at[s, j], ridx)
    pltpu.async_copy(table.at[cidx], rows_v, sem).wait()
    pltpu.sync_copy(rows_v, acc_s.at[ridx], add=True)
    return 0

  lax.fori_loop(0, nch, chunk, 0)


def _p2_body(tk, tq, rows2, cols2, acc_o, ridx, cidx, rows_v, acc_s, sem):
  c = lax.axis_index("c")
  s = lax.axis_index("s")
  _zero_acc(acc_s, rows_v, s)
  plsc.subcore_barrier()

  @pl.when(c == 0)
  def _():
    _p2_run(tk, acc_s, rows2, cols2, ridx, cidx, rows_v, sem, s)

  @pl.when(c == 1)
  def _():
    _p2_run(tq, acc_s, rows2, cols2, ridx, cidx, rows_v, sem, s)

  plsc.subcore_barrier()
  pltpu.sync_copy(acc_s.at[pl.ds(s * RPT, RPT)], acc_o.at[c, pl.ds(s * RPT, RPT)])


def _p2(tk, tq, rows2, cols2):
  f = pl.kernel(
      _p2_body,
      out_type=jax.ShapeDtypeStruct((NC, NP, D), jnp.float32),
      mesh=_mesh,
      scratch_types=[
          pltpu.VMEM((CB,), jnp.int32), pltpu.VMEM((CB,), jnp.int32),
          pltpu.VMEM((CB, D), jnp.float32),
          pltpu.VMEM_SHARED((NP, D), jnp.float32),
          pltpu.SemaphoreType.DMA,
      ])
  return f(tk, tq, rows2, cols2)


def _p4_body(xrq, xt, rows4, cols4, ss_o,
             ridx, cidx, a_v, b_v, cbuf, acc_s, sem, sem2):
  c = lax.axis_index("c")
  s = lax.axis_index("s")
  _zero_acc(acc_s, cbuf, s)
  plsc.subcore_barrier()
  nch = rows4.shape[2]

  def chunk(j, _):
    pltpu.sync_copy(rows4.at[c, s, j], ridx)
    pltpu.sync_copy(cols4.at[c, s, j], cidx)
    d1 = pltpu.async_copy(xrq.at[ridx], a_v, sem)
    d2 = pltpu.async_copy(xt.at[cidx], b_v, sem2)
    d1.wait()
    d2.wait()

    def edge(i, _):
      for g in range(D // 16):
        sl = pl.ds(g * 16, 16)
        cbuf[i, sl] = jnp.abs(a_v[i, sl] - b_v[i, sl])
      return 0

    lax.fori_loop(0, CBP, edge, 0)
    pltpu.sync_copy(cbuf, acc_s.at[ridx], add=True)
    return 0

  lax.fori_loop(0, nch, chunk, 0)
  plsc.subcore_barrier()
  pltpu.sync_copy(acc_s.at[pl.ds(s * RPT, RPT)], ss_o.at[c, pl.ds(s * RPT, RPT)])


def _p4(xrq, xt, rows4, cols4):
  f = pl.kernel(
      _p4_body,
      out_type=jax.ShapeDtypeStruct((NC, NP, D), jnp.float32),
      mesh=_mesh,
      scratch_types=[
          pltpu.VMEM((CBP,), jnp.int32), pltpu.VMEM((CBP,), jnp.int32),
          pltpu.VMEM((CBP, D), jnp.float32), pltpu.VMEM((CBP, D), jnp.float32),
          pltpu.VMEM((CBP, D), jnp.float32),
          pltpu.VMEM_SHARED((NP, D), jnp.float32),
          pltpu.SemaphoreType.DMA, pltpu.SemaphoreType.DMA,
      ])
  return f(xrq, xt, rows4, cols4)


def _k1_body(x_ref, wg_ref, wr_ref, wroot_ref, bgr_ref, xw_ref, gr_ref, fit_ref):
  xb = x_ref[...]
  xw = jnp.dot(xb, wg_ref[...], preferred_element_type=jnp.float32)
  gr = jnp.dot(xb, wr_ref[...], preferred_element_type=jnp.float32)
  groot = jnp.dot(xb, wroot_ref[...], preferred_element_type=jnp.float32)
  xw_ref[...] = xw
  gr_ref[...] = gr
  fit_ref[...] = gr + groot + bgr_ref[0, 0]


def _k1(xp, w_gcn, wg_rel, wg_root, bg_rel):
  grid = NP // BK
  return pl.pallas_call(
      _k1_body,
      grid=(grid,),
      in_specs=[
          pl.BlockSpec((BK, D), lambda i: (i, 0)),
          pl.BlockSpec((D, D), lambda i: (0, 0)),
          pl.BlockSpec((D, 1), lambda i: (0, 0)),
          pl.BlockSpec((D, 1), lambda i: (0, 0)),
          pl.BlockSpec((1, 1), lambda i: (0, 0)),
      ],
      out_specs=[
          pl.BlockSpec((BK, D), lambda i: (i, 0)),
          pl.BlockSpec((BK, 1), lambda i: (i, 0)),
          pl.BlockSpec((BK, 1), lambda i: (i, 0)),
      ],
      out_shape=[
          jax.ShapeDtypeStruct((NP, D), jnp.float32),
          jax.ShapeDtypeStruct((NP, 1), jnp.float32),
          jax.ShapeDtypeStruct((NP, 1), jnp.float32),
      ])(xp, w_gcn, wg_rel, wg_root, bg_rel)


def _k2_body(a1_ref, xw_ref, fit_ref, y_ref, dinv_ref, fitness_ref):
  a = a1_ref[...]
  deg = 1.0 + jnp.sum(a[:, :, 0, :], axis=(0, 1))[:, None]
  agg = jnp.sum(a[:, :, 1, :], axis=(0, 1))[:, None]
  dinv = lax.rsqrt(deg)
  y_ref[...] = dinv * xw_ref[...]
  dinv_ref[...] = dinv
  fitness_ref[...] = jax.nn.sigmoid(agg + fit_ref[...])


def _k2(s1, xw, fit):
  grid = NP // BK
  return pl.pallas_call(
      _k2_body,
      grid=(grid,),
      in_specs=[
          pl.BlockSpec((NC, NS, 3, BK), lambda i: (0, 0, 0, i)),
          pl.BlockSpec((BK, D), lambda i: (i, 0)),
          pl.BlockSpec((BK, 1), lambda i: (i, 0)),
      ],
      out_specs=[
          pl.BlockSpec((BK, D), lambda i: (i, 0)),
          pl.BlockSpec((BK, 1), lambda i: (i, 0)),
          pl.BlockSpec((BK, 1), lambda i: (i, 0)),
      ],
      out_shape=[
          jax.ShapeDtypeStruct((NP, D), jnp.float32),
          jax.ShapeDtypeStruct((NP, 1), jnp.float32),
          jax.ShapeDtypeStruct((NP, 1), jnp.float32),
      ])(s1, xw, fit)


def _leaky(z):
  return jnp.where(z >= 0, z, 0.01 * z)


def _k3_body(t_ref, dinv_ref, xw_ref, bgcn_ref, wk_ref, bk_ref, wq_ref, bq_ref,
             xt_ref, tk_ref, tq_ref, uk_ref, uq_ref):
  t = t_ref[...]
  dinv = dinv_ref[...]
  xt = dinv * (t[0] + t[1]) + (dinv * dinv) * xw_ref[...] + bgcn_ref[...]
  xt_ref[...] = xt
  uk = jnp.exp(_leaky(jnp.dot(xt, wk_ref[...], preferred_element_type=jnp.float32)
                      + bk_ref[0, 0]))
  uq = jnp.exp(_leaky(jnp.dot(xt, wq_ref[...], preferred_element_type=jnp.float32)
                      + bq_ref[0, 0]))
  tk_ref[...] = uk * xt
  tq_ref[...] = uq * xt
  uk_ref[...] = uk
  uq_ref[...] = uq


def _k3(t, dinv, xw, bgcn, wk, bk, wq, bq):
  grid = NP // BK
  return pl.pallas_call(
      _k3_body,
      grid=(grid,),
      in_specs=[
          pl.BlockSpec((NC, BK, D), lambda i: (0, i, 0)),
          pl.BlockSpec((BK, 1), lambda i: (i, 0)),
          pl.BlockSpec((BK, D), lambda i: (i, 0)),
          pl.BlockSpec((1, D), lambda i: (0, 0)),
          pl.BlockSpec((D, 1), lambda i: (0, 0)),
          pl.BlockSpec((1, 1), lambda i: (0, 0)),
          pl.BlockSpec((D, 1), lambda i: (0, 0)),
          pl.BlockSpec((1, 1), lambda i: (0, 0)),
      ],
      out_specs=[
          pl.BlockSpec((BK, D), lambda i: (i, 0)),
          pl.BlockSpec((BK, D), lambda i: (i, 0)),
          pl.BlockSpec((BK, D), lambda i: (i, 0)),
          pl.BlockSpec((BK, 1), lambda i: (i, 0)),
          pl.BlockSpec((BK, 1), lambda i: (i, 0)),
      ],
      out_shape=[
          jax.ShapeDtypeStruct((NP, D), jnp.float32),
          jax.ShapeDtypeStruct((NP, D), jnp.float32),
          jax.ShapeDtypeStruct((NP, D), jnp.float32),
          jax.ShapeDtypeStruct((NP, 1), jnp.float32),
          jax.ShapeDtypeStruct((NP, 1), jnp.float32),
      ])(t, dinv, xw, bgcn, wk, bk, wq, bq)


def _k4_body(p_ref, s2_ref, tk_ref, tq_ref, uk_ref, uq_ref, xt_ref,
             xrq_ref, kerr_ref):
  p = p_ref[...]
  s2 = s2_ref[...]
  tk = tk_ref[...]
  tq = tq_ref[...]
  xt = xt_ref[...]
  dk = jnp.sum(s2[:, :, 0, :], axis=(0, 1))[:, None] + uk_ref[...]
  dq = jnp.sum(s2[:, :, 1, :], axis=(0, 1))[:, None] + uq_ref[...]
  xr_k = (p[0, :, :] + tk) / dk
  xr_q = (p[1, :, :] + tq) / dq
  xrq_ref[...] = xr_q
  kerr_ref[...] = jnp.sum(jnp.abs(xr_k - xt), axis=1, keepdims=True)


def _k4(p2acc, s2, tk, tq, uk, uq, xt):
  grid = NP // BK
  return pl.pallas_call(
      _k4_body,
      grid=(grid,),
      in_specs=[
          pl.BlockSpec((NC, BK, D), lambda i: (0, i, 0)),
          pl.BlockSpec((NC, NS, 2, BK), lambda i: (0, 0, 0, i)),
          pl.BlockSpec((BK, D), lambda i: (i, 0)),
          pl.BlockSpec((BK, D), lambda i: (i, 0)),
          pl.BlockSpec((BK, 1), lambda i: (i, 0)),
          pl.BlockSpec((BK, 1), lambda i: (i, 0)),
          pl.BlockSpec((BK, D), lambda i: (i, 0)),
      ],
      out_specs=[
          pl.BlockSpec((BK, D), lambda i: (i, 0)),
          pl.BlockSpec((BK, 1), lambda i: (i, 0)),
      ],
      out_shape=[
          jax.ShapeDtypeStruct((NP, D), jnp.float32),
          jax.ShapeDtypeStruct((NP, 1), jnp.float32),
      ])(p2acc, s2, tk, tq, uk, uq, xt)


def _k5_body(s1_ref, kerr_ref, ss_ref, fitness_ref, out_ref):
  s1 = s1_ref[...]
  ss = ss_ref[...]
  outdeg = jnp.sum(s1[:, :, 2, :], axis=(0, 1))[:, None]
  sstot = jnp.sum(ss[0] + ss[1], axis=1, keepdims=True)
  out_ref[...] = fitness_ref[...] - 0.1 * (outdeg * kerr_ref[...] - sstot)


def _k5(s1, kerr, ssacc, fitness):
  grid = NP // BK
  return pl.pallas_call(
      _k5_body,
      grid=(grid,),
      in_specs=[
          pl.BlockSpec((NC, NS, 3, BK), lambda i: (0, 0, 0, i)),
          pl.BlockSpec((BK, 1), lambda i: (i, 0)),
          pl.BlockSpec((NC, BK, D), lambda i: (0, i, 0)),
          pl.BlockSpec((BK, 1), lambda i: (i, 0)),
      ],
      out_specs=pl.BlockSpec((BK, 1), lambda i: (i, 0)),
      out_shape=jax.ShapeDtypeStruct((NP, 1), jnp.float32),
      )(s1, kerr, ssacc, fitness)


def _dummy_fill(ea):
  return (N + (jnp.arange(ea, dtype=jnp.int32) % (NP - N))).astype(jnp.int32)


def _pad_edges_split(r, c, e, cb=CB):
  tot = NC * NS * cb
  ch = -(-e // tot)
  ch += ch % 2
  ea = tot * ch
  rp = _dummy_fill(ea).at[:e].set(r)
  cp = _dummy_fill(ea).at[:e].set(c)
  return (rp.reshape(NC, NS, ch, cb), cp.reshape(NC, NS, ch, cb))


def _pad_edges_full(r, c, e):
  tot = NS * CB
  ch = -(-e // tot)
  ea = tot * ch
  rp = _dummy_fill(ea).at[:e].set(r)
  cp = _dummy_fill(ea).at[:e].set(c)
  return (rp.reshape(NS, ch, CB), cp.reshape(NS, ch, CB))


@jax.jit
def kernel(x, edge_index, W_gcn, b_gcn, W_key, b_key, W_query, b_query,
           Wg_rel, bg_rel, Wg_root):
  e = edge_index.shape[1]
  row = edge_index[0].astype(jnp.int32)
  col = edge_index[1].astype(jnp.int32)
  rows3, cols3 = _pad_edges_split(row, col, e)
  rows4, cols4 = _pad_edges_split(row, col, e, CBP)
  rows2, cols2 = _pad_edges_full(row, col, e)

  xp = jnp.pad(x, ((0, NP - N), (0, 0)))
  bgr = bg_rel.reshape(1, 1)
  bkk = b_key.reshape(1, 1)
  bqq = b_query.reshape(1, 1)
  bgcn = b_gcn.reshape(1, D)

  xw, gr1, fit = _k1(xp, W_gcn, Wg_rel, Wg_root, bgr)
  s1 = _scal1(gr1.reshape(-1), rows3, cols3).reshape(NC, NS, 3, NP)
  y, dinv, fitness = _k2(s1, xw, fit)
  t = _gs(y, rows3, cols3)
  xt, tk, tq, uk, uq = _k3(t, dinv, xw, bgcn, W_key, bkk, W_query, bqq)
  p2acc = _p2(tk, tq, rows2, cols2)
  s2 = _scal2(uk.reshape(-1), uq.reshape(-1),
              rows3, cols3).reshape(NC, NS, 2, NP)
  xrq, kerr = _k4(p2acc, s2, tk, tq, uk, uq, xt)
  ssacc = _p4(xrq, xt, rows4, cols4)
  return _k5(s1, kerr, ssacc, fitness).reshape(-1)[:N]
